# baseline (speedup 1.0000x reference)
import functools

import jax
import jax.numpy as jnp
from jax import lax
from jax.experimental import pallas as pl

N_BINS = 64
EPS = 1e-06
LOW = 0.0
HIGH = 1.0


def _tokenize_block(obs_ref, tokens_ref, mask_ref):
    x = obs_ref[...]
    x = jnp.clip(x, LOW + EPS, HIGH - EPS)
    bins = jnp.floor(x * N_BINS).astype(jnp.int32)
    iota = lax.broadcasted_iota(jnp.int32, tokens_ref.shape, 3)
    tokens_ref[...] = (bins[..., None] == iota).astype(jnp.float32)
    mask_ref[...] = jnp.ones(mask_ref.shape, jnp.float32)


@jax.jit
def kernel(observations):
    B, T, D = observations.shape
    bB = 64
    grid = (B // bB,)
    tokens, mask = pl.pallas_call(
        _tokenize_block,
        grid=grid,
        in_specs=[pl.BlockSpec((bB, T, D), lambda i: (i, 0, 0))],
        out_specs=[
            pl.BlockSpec((bB, T, D, N_BINS), lambda i: (i, 0, 0, 0)),
            pl.BlockSpec((bB, T, D), lambda i: (i, 0, 0)),
        ],
        out_shape=[
            jax.ShapeDtypeStruct((B, T, D, N_BINS), jnp.float32),
            jax.ShapeDtypeStruct((B, T, D), jnp.float32),
        ],
    )(observations)
    return (tokens, mask)
